# baseline (speedup 1.0000x reference)
import functools

import jax
import jax.numpy as jnp
from jax import lax
from jax.experimental import pallas as pl
from jax.experimental.pallas import tpu as pltpu
from jax.experimental.pallas import tpu_sc as plsc

_B = 2
_N = 8192
_CIN = 64
_S = 2048
_R2 = 0.2 * 0.2
_NS = 32
_ALPHA = 0.2
_EPS = 1e-5
_DIN = 3 + _CIN
_DPAD = 128
_DOUT = 128
_SBLK = 256
_MBLK = 256


def _fps_body(x_ref, y_ref, z_ref, idx_ref):
    X = x_ref[...]
    Y = y_ref[...]
    Z = z_ref[...]
    rows = lax.broadcasted_iota(jnp.int32, (64, 128), 0)
    cols = lax.broadcasted_iota(jnp.int32, (64, 128), 1)
    iota = rows * 128 + cols
    dists0 = jnp.full((64, 128), 1e10, jnp.float32)

    def body(i, carry):
        dists, far = carry
        idx_ref[0, pl.ds(i, 1), :] = jnp.broadcast_to(far, (1, 8))
        msk = iota == far
        cx = jnp.sum(jnp.where(msk, X, 0.0), keepdims=True)
        cy = jnp.sum(jnp.where(msk, Y, 0.0), keepdims=True)
        cz = jnp.sum(jnp.where(msk, Z, 0.0), keepdims=True)
        dx = X - cx
        dy = Y - cy
        dz = Z - cz
        d = (dx * dx + dy * dy) + dz * dz
        dists = jnp.minimum(dists, d)
        mx = jnp.max(dists, keepdims=True)
        far = jnp.min(jnp.where(dists == mx, iota, _N), keepdims=True)
        return dists, far

    lax.fori_loop(0, _S, body, (dists0, jnp.zeros((1, 1), jnp.int32)))


def _fps(xyz):
    xr = xyz[:, 0, :].reshape(_B * 64, 128)
    yr = xyz[:, 1, :].reshape(_B * 64, 128)
    zr = xyz[:, 2, :].reshape(_B * 64, 128)
    in_spec = pl.BlockSpec((64, 128), lambda b: (b, 0))
    idx = pl.pallas_call(
        _fps_body,
        grid=(_B,),
        in_specs=[in_spec] * 3,
        out_specs=pl.BlockSpec((1, _S, 8), lambda b: (b, 0, 0)),
        out_shape=jax.ShapeDtypeStruct((_B, _S, 8), jnp.int32),
        compiler_params=pltpu.CompilerParams(
            dimension_semantics=("parallel",)),
    )(xr, yr, zr)
    return idx[:, :, 0]


_NCH = _N // 128


def _ballq_body(xn_ref, yn_ref, zn_ref, nxyz_ref, idx_ref):
    xn = xn_ref[0]
    yn = yn_ref[0]
    zn = zn_ref[0]
    nx = nxyz_ref[0]
    xs = nx[:, 0:1].reshape(_SBLK, 1, 1)
    ys = nx[:, 1:2].reshape(_SBLK, 1, 1)
    zs = nx[:, 2:3].reshape(_SBLK, 1, 1)
    s2 = (xs * xs + ys * ys) + zs * zs
    n2 = (xn * xn + yn * yn) + zn * zn
    bf = lambda v: v.astype(jnp.bfloat16).astype(jnp.float32)
    dot = (bf(xs) * bf(xn) + bf(ys) * bf(yn)) + bf(zs) * bf(zn)
    d2 = (s2 + n2) - 2.0 * dot
    maskf = (d2 < _R2).astype(jnp.float32)
    ucol = lax.broadcasted_iota(jnp.int32, (128, 128), 0)
    urow = lax.broadcasted_iota(jnp.int32, (128, 128), 1)
    U128 = (ucol <= urow).astype(jnp.float32)
    W = jnp.dot(maskf.reshape(_SBLK * _NCH, 128), U128,
                preferred_element_type=jnp.float32)
    W3 = W.reshape(_SBLK, _NCH, 128)
    t = W3[:, :, 127:128].reshape(_SBLK, _NCH)
    c1 = lax.broadcasted_iota(jnp.int32, (_NCH, _NCH), 0)
    c2 = lax.broadcasted_iota(jnp.int32, (_NCH, _NCH), 1)
    SU = (c1 < c2).astype(jnp.float32)
    P = jnp.dot(t, SU, preferred_element_type=jnp.float32)
    acc = W3 + P.reshape(_SBLK, _NCH, 1)
    cols = []
    for k in range(_NS):
        ck = jnp.sum((acc <= (k + 0.5)).astype(jnp.float32), axis=(1, 2),
                     keepdims=True)
        cols.append(ck.reshape(_SBLK, 1))
    cnt = jnp.concatenate(cols, axis=1)
    first = cnt[:, 0:1]
    sel = jnp.where(cnt >= _N - 0.5, first, cnt)
    idx_ref[0] = sel.astype(jnp.int32)


def _ballq(xyz, new_xyz):
    xn = xyz[:, 0, :].reshape(_B, 1, _NCH, 128)
    yn = xyz[:, 1, :].reshape(_B, 1, _NCH, 128)
    zn = xyz[:, 2, :].reshape(_B, 1, _NCH, 128)
    coord_spec = pl.BlockSpec((1, 1, _NCH, 128), lambda b, s: (b, 0, 0, 0))
    return pl.pallas_call(
        _ballq_body,
        grid=(_B, _S // _SBLK),
        in_specs=[coord_spec, coord_spec, coord_spec,
                  pl.BlockSpec((1, _SBLK, 3), lambda b, s: (b, s, 0))],
        out_specs=pl.BlockSpec((1, _SBLK, _NS), lambda b, s: (b, s, 0)),
        out_shape=jax.ShapeDtypeStruct((_B, _S, _NS), jnp.int32),
        compiler_params=pltpu.CompilerParams(
            dimension_semantics=("parallel", "parallel")),
    )(xn, yn, zn, new_xyz)


_TOT = _B * _S * _NS + _B * _S
_NCORES = 2
_NSUB = 16
_NW = _NCORES * _NSUB
def _sc_gather(table, idx_all, tot, chunk, nbuf):
    per_w = tot // _NW
    nround = per_w // (chunk * nbuf)
    assert per_w == nround * chunk * nbuf and chunk % 8 == 0 and chunk <= 128
    mesh = plsc.VectorSubcoreMesh(core_axis_name="c", subcore_axis_name="s")

    @functools.partial(
        pl.kernel,
        mesh=mesh,
        out_type=jax.ShapeDtypeStruct((tot, _DPAD), jnp.float32),
        scratch_types=(
            [pltpu.VMEM((chunk,), jnp.int32)] * nbuf
            + [pltpu.VMEM((chunk, _DPAD), jnp.float32)] * nbuf
            + [pltpu.SemaphoreType.DMA] * nbuf
        ),
    )
    def gk(table_hbm, idx_hbm, out_hbm, *scr):
        idx_bufs = scr[:nbuf]
        row_bufs = scr[nbuf:2 * nbuf]
        sems = scr[2 * nbuf:]
        wid = lax.axis_index("s") * _NCORES + lax.axis_index("c")
        base = wid * per_w

        def round_(g, carry):
            offs = [base + (g * nbuf + j) * chunk for j in range(nbuf)]
            for j in range(nbuf):
                pltpu.sync_copy(idx_hbm.at[pl.ds(offs[j], chunk)],
                                idx_bufs[j])
            copies = [
                pltpu.async_copy(table_hbm.at[idx_bufs[j]], row_bufs[j],
                                 sems[j])
                for j in range(nbuf)
            ]
            for j in range(nbuf):
                copies[j].wait()
                pltpu.sync_copy(row_bufs[j],
                                out_hbm.at[pl.ds(offs[j], chunk)])
            return carry

        lax.fori_loop(0, nround, round_, jnp.int32(0))

    return gk(table, idx_all)


def _mlp_attn_body(g_ref, f_ref, w0_ref, w1_ref, w2_ref, s0_ref, h0_ref,
                   s1_ref, h1_ref, s2_ref, h2_ref, ap_ref, ah_ref, out_ref):
    G = g_ref[...]
    F = f_ref[...]
    G3 = G.reshape(_MBLK, _NS, _DPAD)
    F3 = F.reshape(_MBLK, 1, _DPAD)
    colD = lax.broadcasted_iota(jnp.int32, (_MBLK, _NS, _DPAD), 2)
    rel = jnp.where(colD < 3, G3 - F3, G3)
    h = rel.reshape(_MBLK * _NS, _DPAD)
    hf = F
    layers = ((w0_ref, s0_ref, h0_ref),
              (w1_ref, s1_ref, h1_ref),
              (w2_ref, s2_ref, h2_ref))
    for w_ref, sc_ref, sh_ref in layers:
        W = w_ref[...]
        sc = sc_ref[...]
        sh = sh_ref[...]
        h = jnp.maximum(
            jnp.dot(h, W, preferred_element_type=jnp.float32) * sc + sh, 0.0)
        hf = jnp.maximum(
            jnp.dot(hf, W, preferred_element_type=jnp.float32) * sc + sh, 0.0)
    Dg3 = h.reshape(_MBLK, _NS, _DOUT)
    Df3 = hf.reshape(_MBLK, 1, _DOUT)
    dh = Df3 - Dg3
    e = jnp.dot(dh.reshape(_MBLK * _NS, _DOUT), ah_ref[...],
                preferred_element_type=jnp.float32)
    e3 = e.reshape(_MBLK, _NS, _DOUT)
    dp = 2.0 * F3[:, :, 0:3] - G3[:, :, 0:3]
    for c in range(3):
        arow = ap_ref[c, :].reshape(1, 1, _DOUT)
        e3 = e3 + dp[:, :, c:c + 1] * arow
    e3 = jnp.where(e3 >= 0.0, e3, _ALPHA * e3)
    m = jnp.max(e3, axis=1, keepdims=True)
    p = jnp.exp(e3 - m)
    ssum = jnp.sum(p, axis=1, keepdims=True)
    attn = p / ssum
    out_ref[...] = jnp.sum(attn * Dg3, axis=1)


def _mlp_attn(G, F, w0, w1, w2, s0, h0, s1, h1, s2, h2, ap, ah):
    nblk = (_B * _S) // _MBLK
    full = lambda x: pl.BlockSpec(x.shape, lambda i: tuple(0 for _ in x.shape))
    return pl.pallas_call(
        _mlp_attn_body,
        grid=(nblk,),
        in_specs=[
            pl.BlockSpec((_MBLK * _NS, _DPAD), lambda i: (i, 0)),
            pl.BlockSpec((_MBLK, _DPAD), lambda i: (i, 0)),
            full(w0), full(w1), full(w2),
            full(s0), full(h0), full(s1), full(h1), full(s2), full(h2),
            full(ap), full(ah),
        ],
        out_specs=pl.BlockSpec((_MBLK, _DOUT), lambda i: (i, 0)),
        out_shape=jax.ShapeDtypeStruct((_B * _S, _DOUT), jnp.float32),
        compiler_params=pltpu.CompilerParams(
            dimension_semantics=("parallel",)),
    )(G, F, w0, w1, w2, s0, h0, s1, h1, s2, h2, ap, ah)


def kernel(xyz, points, W0, b0, g0, be0, rm0, rv0, W1, b1, g1, be1, rm1, rv1,
           W2, b2, g2, be2, rm2, rv2, a):
    fps_idx = _fps(xyz)
    table = jnp.concatenate(
        [xyz.transpose(0, 2, 1), points.transpose(0, 2, 1),
         jnp.zeros((_B, _N, _DPAD - _DIN), jnp.float32)],
        axis=-1).reshape(_B * _N, _DPAD)
    boff = jnp.arange(_B, dtype=jnp.int32) * _N
    fps_flat = jnp.clip((fps_idx + boff[:, None]).reshape(-1), 0,
                        _B * _N - 1)
    F = _sc_gather(table, fps_flat, _B * _S, 128, 1)
    new_xyz = F[:, :3].reshape(_B, _S, 3)

    idx = _ballq(xyz, new_xyz)
    idx_flat = jnp.clip((idx + boff[:, None, None]).reshape(-1), 0,
                        _B * _N - 1)
    G = _sc_gather(table, idx_flat, _B * _S * _NS, 128, 4)

    def prep(W, b, g, be, rm, rv, cin_pad):
        sc = g / jnp.sqrt(rv + _EPS)
        sh = (b - rm) * sc + be
        Wt = W.T
        if cin_pad > Wt.shape[0]:
            Wt = jnp.concatenate(
                [Wt, jnp.zeros((cin_pad - Wt.shape[0], Wt.shape[1]),
                               jnp.float32)], axis=0)
        return Wt, sc.reshape(1, -1), sh.reshape(1, -1)

    w0p, s0, sh0 = prep(W0, b0, g0, be0, rm0, rv0, _DPAD)
    w1p, s1, sh1 = prep(W1, b1, g1, be1, rm1, rv1, 64)
    w2p, s2, sh2 = prep(W2, b2, g2, be2, rm2, rv2, 64)
    ap = jnp.concatenate([a[:3], jnp.zeros((5, _DOUT), jnp.float32)], axis=0)
    ah = a[3:]

    gp = _mlp_attn(G, F, w0p, w1p, w2p, s0, sh0, s1, sh1, s2, sh2, ap, ah)

    new_xyz_out = new_xyz.transpose(0, 2, 1)
    gp_out = gp.reshape(_B, _S, _DOUT).transpose(0, 2, 1)
    return (new_xyz_out, gp_out)


# speedup vs baseline: 1.6777x; 1.6777x over previous
import functools

import jax
import jax.numpy as jnp
from jax import lax
from jax.experimental import pallas as pl
from jax.experimental.pallas import tpu as pltpu
from jax.experimental.pallas import tpu_sc as plsc

_B = 2
_N = 8192
_CIN = 64
_S = 2048
_R2 = 0.2 * 0.2
_NS = 32
_ALPHA = 0.2
_EPS = 1e-5
_DIN = 3 + _CIN
_DPAD = 128
_DOUT = 128
_SBLK = 256
_MBLK = 256


def _fps_body(x_ref, y_ref, z_ref, idx_ref):
    X = x_ref[...]
    Y = y_ref[...]
    Z = z_ref[...]
    rows = lax.broadcasted_iota(jnp.int32, (1, 64, 128), 1)
    cols = lax.broadcasted_iota(jnp.int32, (1, 64, 128), 2)
    iota = rows * 128 + cols
    dists0 = jnp.full((_B, 64, 128), 1e10, jnp.float32)

    def body(i, carry):
        dists, far = carry
        idx_ref[pl.ds(i, 1), :] = far.reshape(1, _B)
        msk = iota == far
        cx = jnp.sum(jnp.where(msk, X, 0.0), axis=(1, 2), keepdims=True)
        cy = jnp.sum(jnp.where(msk, Y, 0.0), axis=(1, 2), keepdims=True)
        cz = jnp.sum(jnp.where(msk, Z, 0.0), axis=(1, 2), keepdims=True)
        dx = X - cx
        dy = Y - cy
        dz = Z - cz
        d = (dx * dx + dy * dy) + dz * dz
        dists = jnp.minimum(dists, d)
        mx = jnp.max(dists, axis=(1, 2), keepdims=True)
        far = jnp.min(jnp.where(dists == mx, iota, _N), axis=(1, 2),
                      keepdims=True)
        return dists, far

    lax.fori_loop(0, _S, body, (dists0, jnp.zeros((_B, 1, 1), jnp.int32)))


def _fps(xyz):
    xr = xyz[:, 0, :].reshape(_B, 64, 128)
    yr = xyz[:, 1, :].reshape(_B, 64, 128)
    zr = xyz[:, 2, :].reshape(_B, 64, 128)
    full3 = pl.BlockSpec((_B, 64, 128), lambda: (0, 0, 0))
    idx = pl.pallas_call(
        _fps_body,
        in_specs=[full3] * 3,
        out_specs=pl.BlockSpec((_S, _B), lambda: (0, 0)),
        out_shape=jax.ShapeDtypeStruct((_S, _B), jnp.int32),
    )(xr, yr, zr)
    return idx.T


_NCH = _N // 128


def _ballq_body(xn_ref, yn_ref, zn_ref, nxyz_ref, idx_ref):
    xn = xn_ref[0]
    yn = yn_ref[0]
    zn = zn_ref[0]
    nx = nxyz_ref[0]
    xs = nx[:, 0:1].reshape(_SBLK, 1, 1)
    ys = nx[:, 1:2].reshape(_SBLK, 1, 1)
    zs = nx[:, 2:3].reshape(_SBLK, 1, 1)
    s2 = (xs * xs + ys * ys) + zs * zs
    n2 = (xn * xn + yn * yn) + zn * zn
    bf = lambda v: v.astype(jnp.bfloat16).astype(jnp.float32)
    dot = (bf(xs) * bf(xn) + bf(ys) * bf(yn)) + bf(zs) * bf(zn)
    d2 = (s2 + n2) - 2.0 * dot
    maskf = (d2 < _R2).astype(jnp.float32)
    ucol = lax.broadcasted_iota(jnp.int32, (128, 128), 0)
    urow = lax.broadcasted_iota(jnp.int32, (128, 128), 1)
    U128 = (ucol <= urow).astype(jnp.float32)
    W = jnp.dot(maskf.reshape(_SBLK * _NCH, 128), U128,
                preferred_element_type=jnp.float32)
    W3 = W.reshape(_SBLK, _NCH, 128)
    t = W3[:, :, 127:128].reshape(_SBLK, _NCH)
    c1 = lax.broadcasted_iota(jnp.int32, (_NCH, _NCH), 0)
    c2 = lax.broadcasted_iota(jnp.int32, (_NCH, _NCH), 1)
    SU = (c1 < c2).astype(jnp.float32)
    P = jnp.dot(t, SU, preferred_element_type=jnp.float32)
    acc = W3 + P.reshape(_SBLK, _NCH, 1)
    cols = []
    for k in range(_NS):
        ck = jnp.sum((acc <= (k + 0.5)).astype(jnp.float32), axis=(1, 2),
                     keepdims=True)
        cols.append(ck.reshape(_SBLK, 1))
    cnt = jnp.concatenate(cols, axis=1)
    first = cnt[:, 0:1]
    sel = jnp.where(cnt >= _N - 0.5, first, cnt)
    idx_ref[0] = sel.astype(jnp.int32)


def _ballq(xyz, new_xyz):
    xn = xyz[:, 0, :].reshape(_B, 1, _NCH, 128)
    yn = xyz[:, 1, :].reshape(_B, 1, _NCH, 128)
    zn = xyz[:, 2, :].reshape(_B, 1, _NCH, 128)
    coord_spec = pl.BlockSpec((1, 1, _NCH, 128), lambda b, s: (b, 0, 0, 0))
    return pl.pallas_call(
        _ballq_body,
        grid=(_B, _S // _SBLK),
        in_specs=[coord_spec, coord_spec, coord_spec,
                  pl.BlockSpec((1, _SBLK, 3), lambda b, s: (b, s, 0))],
        out_specs=pl.BlockSpec((1, _SBLK, _NS), lambda b, s: (b, s, 0)),
        out_shape=jax.ShapeDtypeStruct((_B, _S, _NS), jnp.int32),
    )(xn, yn, zn, new_xyz)


_TOT = _B * _S * _NS + _B * _S
_NCORES = 2
_NSUB = 16
_NW = _NCORES * _NSUB
def _sc_gather(table, idx_all, tot, chunk, nbuf):
    per_w = tot // _NW
    nround = per_w // (chunk * nbuf)
    assert per_w == nround * chunk * nbuf and chunk % 8 == 0 and chunk <= 128
    mesh = plsc.VectorSubcoreMesh(core_axis_name="c", subcore_axis_name="s")

    @functools.partial(
        pl.kernel,
        mesh=mesh,
        out_type=jax.ShapeDtypeStruct((tot, _DPAD), jnp.float32),
        scratch_types=(
            [pltpu.VMEM((chunk,), jnp.int32)] * nbuf
            + [pltpu.VMEM((chunk, _DPAD), jnp.float32)] * nbuf
            + [pltpu.SemaphoreType.DMA] * nbuf
        ),
    )
    def gk(table_hbm, idx_hbm, out_hbm, *scr):
        idx_bufs = scr[:nbuf]
        row_bufs = scr[nbuf:2 * nbuf]
        sems = scr[2 * nbuf:]
        wid = lax.axis_index("s") * _NCORES + lax.axis_index("c")
        base = wid * per_w

        def round_(g, carry):
            offs = [base + (g * nbuf + j) * chunk for j in range(nbuf)]
            for j in range(nbuf):
                pltpu.sync_copy(idx_hbm.at[pl.ds(offs[j], chunk)],
                                idx_bufs[j])
            copies = [
                pltpu.async_copy(table_hbm.at[idx_bufs[j]], row_bufs[j],
                                 sems[j])
                for j in range(nbuf)
            ]
            for j in range(nbuf):
                copies[j].wait()
                pltpu.sync_copy(row_bufs[j],
                                out_hbm.at[pl.ds(offs[j], chunk)])
            return carry

        lax.fori_loop(0, nround, round_, jnp.int32(0))

    return gk(table, idx_all)


def _mlp_attn_body(g_ref, f_ref, w0_ref, w1_ref, w2_ref, s0_ref, h0_ref,
                   s1_ref, h1_ref, s2_ref, h2_ref, ap_ref, ah_ref, out_ref):
    G = g_ref[...]
    F = f_ref[...]
    G3 = G.reshape(_MBLK, _NS, _DPAD)
    F3 = F.reshape(_MBLK, 1, _DPAD)
    colD = lax.broadcasted_iota(jnp.int32, (_MBLK, _NS, _DPAD), 2)
    rel = jnp.where(colD < 3, G3 - F3, G3)
    h = rel.reshape(_MBLK * _NS, _DPAD)
    hf = F
    layers = ((w0_ref, s0_ref, h0_ref),
              (w1_ref, s1_ref, h1_ref),
              (w2_ref, s2_ref, h2_ref))
    for w_ref, sc_ref, sh_ref in layers:
        W = w_ref[...]
        sc = sc_ref[...]
        sh = sh_ref[...]
        h = jnp.maximum(
            jnp.dot(h, W, preferred_element_type=jnp.float32) * sc + sh, 0.0)
        hf = jnp.maximum(
            jnp.dot(hf, W, preferred_element_type=jnp.float32) * sc + sh, 0.0)
    Dg3 = h.reshape(_MBLK, _NS, _DOUT)
    Df3 = hf.reshape(_MBLK, 1, _DOUT)
    dh = Df3 - Dg3
    e = jnp.dot(dh.reshape(_MBLK * _NS, _DOUT), ah_ref[...],
                preferred_element_type=jnp.float32)
    e3 = e.reshape(_MBLK, _NS, _DOUT)
    dp = 2.0 * F3[:, :, 0:3] - G3[:, :, 0:3]
    for c in range(3):
        arow = ap_ref[c, :].reshape(1, 1, _DOUT)
        e3 = e3 + dp[:, :, c:c + 1] * arow
    e3 = jnp.where(e3 >= 0.0, e3, _ALPHA * e3)
    m = jnp.max(e3, axis=1, keepdims=True)
    p = jnp.exp(e3 - m)
    ssum = jnp.sum(p, axis=1, keepdims=True)
    attn = p / ssum
    out_ref[...] = jnp.sum(attn * Dg3, axis=1)


def _mlp_attn(G, F, w0, w1, w2, s0, h0, s1, h1, s2, h2, ap, ah):
    nblk = (_B * _S) // _MBLK
    full = lambda x: pl.BlockSpec(x.shape, lambda i: tuple(0 for _ in x.shape))
    return pl.pallas_call(
        _mlp_attn_body,
        grid=(nblk,),
        in_specs=[
            pl.BlockSpec((_MBLK * _NS, _DPAD), lambda i: (i, 0)),
            pl.BlockSpec((_MBLK, _DPAD), lambda i: (i, 0)),
            full(w0), full(w1), full(w2),
            full(s0), full(h0), full(s1), full(h1), full(s2), full(h2),
            full(ap), full(ah),
        ],
        out_specs=pl.BlockSpec((_MBLK, _DOUT), lambda i: (i, 0)),
        out_shape=jax.ShapeDtypeStruct((_B * _S, _DOUT), jnp.float32),
    )(G, F, w0, w1, w2, s0, h0, s1, h1, s2, h2, ap, ah)


def kernel(xyz, points, W0, b0, g0, be0, rm0, rv0, W1, b1, g1, be1, rm1, rv1,
           W2, b2, g2, be2, rm2, rv2, a):
    fps_idx = _fps(xyz)
    table = jnp.concatenate(
        [xyz.transpose(0, 2, 1), points.transpose(0, 2, 1),
         jnp.zeros((_B, _N, _DPAD - _DIN), jnp.float32)],
        axis=-1).reshape(_B * _N, _DPAD)
    boff = jnp.arange(_B, dtype=jnp.int32) * _N
    fps_flat = jnp.clip((fps_idx + boff[:, None]).reshape(-1), 0,
                        _B * _N - 1)
    F = _sc_gather(table, fps_flat, _B * _S, 128, 1)
    new_xyz = F[:, :3].reshape(_B, _S, 3)

    idx = _ballq(xyz, new_xyz)
    idx_flat = jnp.clip((idx + boff[:, None, None]).reshape(-1), 0,
                        _B * _N - 1)
    G = _sc_gather(table, idx_flat, _B * _S * _NS, 128, 4)

    def prep(W, b, g, be, rm, rv, cin_pad):
        sc = g / jnp.sqrt(rv + _EPS)
        sh = (b - rm) * sc + be
        Wt = W.T
        if cin_pad > Wt.shape[0]:
            Wt = jnp.concatenate(
                [Wt, jnp.zeros((cin_pad - Wt.shape[0], Wt.shape[1]),
                               jnp.float32)], axis=0)
        return Wt, sc.reshape(1, -1), sh.reshape(1, -1)

    w0p, s0, sh0 = prep(W0, b0, g0, be0, rm0, rv0, _DPAD)
    w1p, s1, sh1 = prep(W1, b1, g1, be1, rm1, rv1, 64)
    w2p, s2, sh2 = prep(W2, b2, g2, be2, rm2, rv2, 64)
    ap = jnp.concatenate([a[:3], jnp.zeros((5, _DOUT), jnp.float32)], axis=0)
    ah = a[3:]

    gp = _mlp_attn(G, F, w0p, w1p, w2p, s0, sh0, s1, sh1, s2, sh2, ap, ah)

    new_xyz_out = new_xyz.transpose(0, 2, 1)
    gp_out = gp.reshape(_B, _S, _DOUT).transpose(0, 2, 1)
    return (new_xyz_out, gp_out)
